# baseline (speedup 1.0000x reference)
import functools

import jax
import jax.numpy as jnp
from jax import lax
from jax.experimental import pallas as pl
from jax.experimental.pallas import tpu as pltpu
from jax.experimental.pallas import tpu_sc as plsc

_NMS_THR = 0.5
_SCORE_THR = 0.05
_MAX_DET = 300
_NEG = -1e30
_OUT_PAD = 320
_CHUNK = 80


def _prep_body(cls_ref, box_ref, s_ref, x1_ref, y1_ref, x2_ref, y2_ref):
  cls = cls_ref[...]
  m = jnp.max(cls, axis=1)
  s_ref[...] = jnp.where(m > _SCORE_THR, m, _NEG)[None, None, :]
  box = box_ref[...]
  x1_ref[...] = box[None, None, :, 0]
  y1_ref[...] = box[None, None, :, 1]
  x2_ref[...] = box[None, None, :, 2]
  y2_ref[...] = box[None, None, :, 3]


def _nms_body(s_in_ref, x1_ref, y1_ref, x2_ref, y2_ref,
              ox1_ref, oy1_ref, ox2_ref, oy2_ref,
              osc_ref, oidx_ref, oval_ref,
              s_ref):
  B, N = s_in_ref.shape
  s_ref[...] = s_in_ref[...]
  iota_n = lax.broadcasted_iota(jnp.int32, (B, N), 1).astype(jnp.float32)
  iota_o = lax.broadcasted_iota(jnp.int32, (B, _OUT_PAD), 1)
  for r in (ox1_ref, oy1_ref, ox2_ref, oy2_ref, osc_ref, oval_ref):
    r[...] = jnp.full((B, _OUT_PAD), -1.0, jnp.float32)
  oidx_ref[...] = jnp.zeros((B, _OUT_PAD), jnp.float32)

  def body(i, carry):
    s = s_ref[...]
    best = jnp.max(s, axis=1, keepdims=True)
    valid = best > (_NEG / 2)
    bidx = jnp.min(jnp.where(s == best, iota_n, 1e9), axis=1, keepdims=True)
    pos = iota_n == bidx

    def pick(a):
      return jnp.max(jnp.where(pos, a, -jnp.inf), axis=1, keepdims=True)

    x1, y1, x2, y2 = x1_ref[...], y1_ref[...], x2_ref[...], y2_ref[...]
    bx1, by1, bx2, by2 = pick(x1), pick(y1), pick(x2), pick(y2)
    barea = (bx2 - bx1) * (by2 - by1)

    xx1 = jnp.maximum(bx1, x1)
    yy1 = jnp.maximum(by1, y1)
    xx2 = jnp.minimum(bx2, x2)
    yy2 = jnp.minimum(by2, y2)
    inter = jnp.maximum(xx2 - xx1, 0.0) * jnp.maximum(yy2 - yy1, 0.0)
    area = (x2 - x1) * (y2 - y1)
    iou = inter / jnp.maximum(barea + area - inter, 1e-8)
    suppress = (iou > _NMS_THR) | pos
    s_ref[...] = jnp.where(suppress & valid, _NEG, s)

    col = iota_o == i
    colv = col & valid
    ox1_ref[...] = jnp.where(colv, bx1, ox1_ref[...])
    oy1_ref[...] = jnp.where(colv, by1, oy1_ref[...])
    ox2_ref[...] = jnp.where(colv, bx2, ox2_ref[...])
    oy2_ref[...] = jnp.where(colv, by2, oy2_ref[...])
    osc_ref[...] = jnp.where(colv, best, osc_ref[...])
    oidx_ref[...] = jnp.where(col, bidx, oidx_ref[...])
    oval_ref[...] = jnp.where(colv, 1.0, oval_ref[...])
    return carry

  lax.fori_loop(0, _MAX_DET, body, 0)


def _rowwise_arg_max(rows_v, ncols, r0, iota16):
  rows16 = iota16 + r0

  def col(j, st):
    m, marg = st
    g = plsc.load_gather(rows_v, [rows16, jnp.full((16,), 0, jnp.int32) + j])
    upd = g > m
    m = jnp.where(upd, g, m)
    marg = jnp.where(upd, jnp.full((16,), 0.0) + j.astype(jnp.float32), marg)
    return m, marg

  init = (jnp.full((16,), -jnp.inf, jnp.float32), jnp.zeros((16,), jnp.float32))
  return lax.fori_loop(0, ncols, col, init)


def _sc_sel_body(cls_hbm, rel_hbm, idx_hbm, val_hbm,
                 olab_hbm, ops_hbm, opl_hbm,
                 idx_v, val_v, idxg_v, idxe_v, crows_v, rbuf_v,
                 olab_v, ops_v, opl_v, sem):
  N = cls_hbm.shape[0] // 8
  C = cls_hbm.shape[1]
  R = rel_hbm.shape[0] // (8 * N)
  wid = lax.axis_index("s") * 2 + lax.axis_index("c")
  b = wid // 4
  off = b * _OUT_PAD + (wid % 4) * _CHUNK
  iota16 = lax.iota(jnp.int32, 16)

  pltpu.sync_copy(idx_hbm.at[pl.ds(off, _CHUNK)], idx_v)
  pltpu.sync_copy(val_hbm.at[pl.ds(off, _CHUNK)], val_v)
  for j in range(0, _CHUNK, 16):
    iv = idx_v[pl.ds(j, 16)].astype(jnp.int32) + b * N
    idxg_v[pl.ds(j, 16)] = iv
  pltpu.async_copy(cls_hbm.at[idxg_v], crows_v, sem).wait()

  def fill_idx(j, carry):
    for k in range(0, _CHUNK, 16):
      base = idxg_v[pl.ds(k, 16)] * R + j
      idxe_v[pl.ds(j * _CHUNK + k, 16)] = base
    return carry
  lax.fori_loop(0, R, fill_idx, 0)

  def fire(j, carry):
    pltpu.async_copy(rel_hbm.at[idxe_v.at[pl.ds(j * _CHUNK, _CHUNK)]],
                     rbuf_v.at[pl.ds(j * _CHUNK, _CHUNK)], sem)
    return carry
  lax.fori_loop(0, R, fire, 0)
  pltpu.make_async_copy(rel_hbm.at[pl.ds(0, R * _CHUNK)], rbuf_v, sem).wait()

  for r0 in range(0, _CHUNK, 16):
    vv = val_v[pl.ds(r0, 16)] > 0.0
    _, clab = _rowwise_arg_max(crows_v, C, r0, iota16)

    def rcol(j, st):
      m, marg = st
      g = rbuf_v[pl.ds(j * _CHUNK + r0, 16)]
      upd = g > m
      m = jnp.where(upd, g, m)
      marg = jnp.where(upd, jnp.full((16,), 0.0) + j.astype(jnp.float32), marg)
      return m, marg
    rmax, rlab = lax.fori_loop(
        0, R, rcol,
        (jnp.full((16,), -jnp.inf, jnp.float32), jnp.zeros((16,), jnp.float32)))

    olab_v[pl.ds(r0, 16)] = jnp.where(vv, clab, -1.0)
    ops_v[pl.ds(r0, 16)] = jnp.where(vv, rmax, -1.0)
    opl_v[pl.ds(r0, 16)] = jnp.where(vv, rlab, -1.0)

  pltpu.sync_copy(olab_v, olab_hbm.at[pl.ds(off, _CHUNK)])
  pltpu.sync_copy(ops_v, ops_hbm.at[pl.ds(off, _CHUNK)])
  pltpu.sync_copy(opl_v, opl_hbm.at[pl.ds(off, _CHUNK)])


def kernel(boxes, classification, relationship):
  B, N, C = classification.shape
  R = relationship.shape[2]
  NB = 2000
  G = N // NB
  fbn = jax.ShapeDtypeStruct((B * G, 1, NB), jnp.float32)

  planes = pl.pallas_call(
      _prep_body,
      grid=(B * G,),
      in_specs=[
          pl.BlockSpec((NB, C), lambda i: (i, 0)),
          pl.BlockSpec((NB, 4), lambda i: (i, 0)),
      ],
      out_specs=[pl.BlockSpec((1, 1, NB), lambda i: (i, 0, 0))] * 5,
      out_shape=[fbn] * 5,
  )(classification.reshape(B * N, C), boxes.reshape(B * N, 4))
  s, x1, y1, x2, y2 = [p.reshape(B, N) for p in planes]

  fout = jax.ShapeDtypeStruct((B, _OUT_PAD), jnp.float32)
  nouts = pl.pallas_call(
      _nms_body,
      out_shape=[fout] * 7,
      scratch_shapes=[pltpu.VMEM((B, N), jnp.float32)],
  )(s, x1, y1, x2, y2)
  ox1, oy1, ox2, oy2, osc, oidx, oval = nouts

  mesh = plsc.VectorSubcoreMesh(core_axis_name="c", subcore_axis_name="s")
  sc_sel = functools.partial(
      pl.kernel,
      mesh=mesh,
      out_type=[jax.ShapeDtypeStruct((B * _OUT_PAD,), jnp.float32)] * 3,
      compiler_params=pltpu.CompilerParams(needs_layout_passes=False, use_tc_tiling_on_sc=False),
      scratch_types=[
          pltpu.VMEM((_CHUNK,), jnp.float32),
          pltpu.VMEM((_CHUNK,), jnp.float32),
          pltpu.VMEM((_CHUNK,), jnp.int32),
          pltpu.VMEM((R * _CHUNK,), jnp.int32),
          pltpu.VMEM((_CHUNK, C), jnp.float32),
          pltpu.VMEM((R * _CHUNK,), jnp.float32),
          pltpu.VMEM((_CHUNK,), jnp.float32),
          pltpu.VMEM((_CHUNK,), jnp.float32),
          pltpu.VMEM((_CHUNK,), jnp.float32),
          pltpu.SemaphoreType.DMA,
      ],
  )(_sc_sel_body)
  olab, ops_, opl = [o.reshape(B, _OUT_PAD) for o in sc_sel(
      classification.reshape(B * N, C), relationship.reshape(B * N * R),
      oidx.reshape(B * _OUT_PAD), oval.reshape(B * _OUT_PAD))]

  boxes_out = jnp.stack([ox1[:, :_MAX_DET], oy1[:, :_MAX_DET],
                         ox2[:, :_MAX_DET], oy2[:, :_MAX_DET]], axis=-1)
  return (boxes_out, osc[:, :_MAX_DET], olab[:, :_MAX_DET].astype(jnp.int32),
          ops_[:, :_MAX_DET], opl[:, :_MAX_DET].astype(jnp.int32))


# speedup vs baseline: 1.1591x; 1.1591x over previous
import functools

import jax
import jax.numpy as jnp
from jax import lax
from jax.experimental import pallas as pl
from jax.experimental.pallas import tpu as pltpu
from jax.experimental.pallas import tpu_sc as plsc

_NMS_THR = 0.5
_SCORE_THR = 0.05
_MAX_DET = 300
_NEG = -1e30
_OUT_PAD = 320
_CHUNK = 80


def _prep_body(cls_ref, box_ref, s_ref, lab_ref, x1_ref, y1_ref, x2_ref, y2_ref):
  cls = cls_ref[...]
  m = jnp.max(cls, axis=2)
  iota_c = lax.broadcasted_iota(jnp.int32, cls.shape, 2).astype(jnp.float32)
  lab = jnp.min(jnp.where(cls == m[:, :, None], iota_c, 1e9), axis=2)
  s_ref[...] = jnp.where(m > _SCORE_THR, m, _NEG)[:, None, None, :]
  lab_ref[...] = lab[:, None, None, :]
  box = box_ref[...]
  x1_ref[...] = box[:, None, None, :, 0]
  y1_ref[...] = box[:, None, None, :, 1]
  x2_ref[...] = box[:, None, None, :, 2]
  y2_ref[...] = box[:, None, None, :, 3]


def _nms_body(s_in_ref, x1_ref, y1_ref, x2_ref, y2_ref,
              ox1_ref, oy1_ref, ox2_ref, oy2_ref,
              osc_ref, oidx_ref, oval_ref,
              s_ref):
  B, N = s_in_ref.shape
  s_ref[...] = s_in_ref[...]
  iota_n = lax.broadcasted_iota(jnp.int32, (B, N), 1).astype(jnp.float32)
  iota_o = lax.broadcasted_iota(jnp.int32, (B, _OUT_PAD), 1)
  for r in (ox1_ref, oy1_ref, ox2_ref, oy2_ref, osc_ref, oval_ref):
    r[...] = jnp.full((B, _OUT_PAD), -1.0, jnp.float32)
  oidx_ref[...] = jnp.zeros((B, _OUT_PAD), jnp.float32)

  def body(i, carry):
    s = s_ref[...]
    best = jnp.max(s, axis=1, keepdims=True)
    valid = best > (_NEG / 2)
    bidx = jnp.min(jnp.where(s == best, iota_n, 1e9), axis=1, keepdims=True)
    pos = iota_n == bidx

    def pick(a):
      return jnp.max(jnp.where(pos, a, -jnp.inf), axis=1, keepdims=True)

    x1, y1, x2, y2 = x1_ref[...], y1_ref[...], x2_ref[...], y2_ref[...]
    bx1, by1, bx2, by2 = pick(x1), pick(y1), pick(x2), pick(y2)
    barea = (bx2 - bx1) * (by2 - by1)

    xx1 = jnp.maximum(bx1, x1)
    yy1 = jnp.maximum(by1, y1)
    xx2 = jnp.minimum(bx2, x2)
    yy2 = jnp.minimum(by2, y2)
    inter = jnp.maximum(xx2 - xx1, 0.0) * jnp.maximum(yy2 - yy1, 0.0)
    area = (x2 - x1) * (y2 - y1)
    iou = inter / jnp.maximum(barea + area - inter, 1e-8)
    suppress = (iou > _NMS_THR) | pos
    s_ref[...] = jnp.where(suppress & valid, _NEG, s)

    col = iota_o == i
    colv = col & valid
    ox1_ref[...] = jnp.where(colv, bx1, ox1_ref[...])
    oy1_ref[...] = jnp.where(colv, by1, oy1_ref[...])
    ox2_ref[...] = jnp.where(colv, bx2, ox2_ref[...])
    oy2_ref[...] = jnp.where(colv, by2, oy2_ref[...])
    osc_ref[...] = jnp.where(colv, best, osc_ref[...])
    oidx_ref[...] = jnp.where(col, bidx, oidx_ref[...])
    oval_ref[...] = jnp.where(colv, 1.0, oval_ref[...])
    return carry

  lax.fori_loop(0, _MAX_DET, body, 0)


def _rowwise_arg_max(rows_v, ncols, r0, iota16):
  rows16 = iota16 + r0

  def col(j, st):
    m, marg = st
    g = plsc.load_gather(rows_v, [rows16, jnp.full((16,), 0, jnp.int32) + j])
    upd = g > m
    m = jnp.where(upd, g, m)
    marg = jnp.where(upd, jnp.full((16,), 0.0) + j.astype(jnp.float32), marg)
    return m, marg

  init = (jnp.full((16,), -jnp.inf, jnp.float32), jnp.zeros((16,), jnp.float32))
  return lax.fori_loop(0, ncols, col, init)


def _sc_sel_body(lab_hbm, rel_hbm, idx_hbm, val_hbm,
                 olab_hbm, ops_hbm, opl_hbm,
                 idx_v, val_v, idxg_v, idxe_v, plane_v, rbuf_v,
                 olab_v, ops_v, opl_v, sem):
  N = lab_hbm.shape[1]
  R = rel_hbm.shape[0] // (8 * N)
  wid = lax.axis_index("s") * 2 + lax.axis_index("c")
  b = wid // 4
  off = b * _OUT_PAD + (wid % 4) * _CHUNK
  iota16 = lax.iota(jnp.int32, 16)

  pltpu.sync_copy(idx_hbm.at[pl.ds(off, _CHUNK)], idx_v)
  pltpu.sync_copy(val_hbm.at[pl.ds(off, _CHUNK)], val_v)
  for j in range(0, _CHUNK, 16):
    iv = idx_v[pl.ds(j, 16)].astype(jnp.int32) + b * N
    idxg_v[pl.ds(j, 16)] = iv
  pltpu.sync_copy(lab_hbm.at[b], plane_v)

  def fill_idx(j, carry):
    for k in range(0, _CHUNK, 16):
      base = idxg_v[pl.ds(k, 16)] * R + j
      idxe_v[pl.ds(j * _CHUNK + k, 16)] = base
    return carry
  lax.fori_loop(0, R, fill_idx, 0)

  def fire(j, carry):
    pltpu.async_copy(rel_hbm.at[idxe_v.at[pl.ds(j * _CHUNK, _CHUNK)]],
                     rbuf_v.at[pl.ds(j * _CHUNK, _CHUNK)], sem)
    return carry
  lax.fori_loop(0, R, fire, 0)
  pltpu.make_async_copy(rel_hbm.at[pl.ds(0, R * _CHUNK)], rbuf_v, sem).wait()

  for r0 in range(0, _CHUNK, 16):
    vv = val_v[pl.ds(r0, 16)] > 0.0
    clab = plsc.load_gather(plane_v, [idx_v[pl.ds(r0, 16)].astype(jnp.int32)])

    def rcol(j, st):
      m, marg = st
      g = rbuf_v[pl.ds(j * _CHUNK + r0, 16)]
      upd = g > m
      m = jnp.where(upd, g, m)
      marg = jnp.where(upd, jnp.full((16,), 0.0) + j.astype(jnp.float32), marg)
      return m, marg
    rmax, rlab = lax.fori_loop(
        0, R, rcol,
        (jnp.full((16,), -jnp.inf, jnp.float32), jnp.zeros((16,), jnp.float32)))

    olab_v[pl.ds(r0, 16)] = jnp.where(vv, clab, -1.0)
    ops_v[pl.ds(r0, 16)] = jnp.where(vv, rmax, -1.0)
    opl_v[pl.ds(r0, 16)] = jnp.where(vv, rlab, -1.0)

  pltpu.sync_copy(olab_v, olab_hbm.at[pl.ds(off, _CHUNK)])
  pltpu.sync_copy(ops_v, ops_hbm.at[pl.ds(off, _CHUNK)])
  pltpu.sync_copy(opl_v, opl_hbm.at[pl.ds(off, _CHUNK)])


def kernel(boxes, classification, relationship):
  B, N, C = classification.shape
  R = relationship.shape[2]
  NB = 2000
  G = N // NB
  fbn = jax.ShapeDtypeStruct((B, G, 1, NB), jnp.float32)

  planes = pl.pallas_call(
      _prep_body,
      grid=(B, G),
      in_specs=[
          pl.BlockSpec((1, NB, C), lambda b, i: (b, i, 0)),
          pl.BlockSpec((1, NB, 4), lambda b, i: (b, i, 0)),
      ],
      out_specs=[pl.BlockSpec((1, 1, 1, NB), lambda b, i: (b, i, 0, 0))] * 6,
      out_shape=[fbn] * 6,
  )(classification, boxes)
  s, lab, x1, y1, x2, y2 = [p.reshape(B, N) for p in planes]

  fout = jax.ShapeDtypeStruct((B, _OUT_PAD), jnp.float32)
  nouts = pl.pallas_call(
      _nms_body,
      out_shape=[fout] * 7,
      scratch_shapes=[pltpu.VMEM((B, N), jnp.float32)],
  )(s, x1, y1, x2, y2)
  ox1, oy1, ox2, oy2, osc, oidx, oval = nouts

  mesh = plsc.VectorSubcoreMesh(core_axis_name="c", subcore_axis_name="s")
  sc_sel = functools.partial(
      pl.kernel,
      mesh=mesh,
      out_type=[jax.ShapeDtypeStruct((B * _OUT_PAD,), jnp.float32)] * 3,
      compiler_params=pltpu.CompilerParams(needs_layout_passes=False, use_tc_tiling_on_sc=False),
      scratch_types=[
          pltpu.VMEM((_CHUNK,), jnp.float32),
          pltpu.VMEM((_CHUNK,), jnp.float32),
          pltpu.VMEM((_CHUNK,), jnp.int32),
          pltpu.VMEM((R * _CHUNK,), jnp.int32),
          pltpu.VMEM((N,), jnp.float32),
          pltpu.VMEM((R * _CHUNK,), jnp.float32),
          pltpu.VMEM((_CHUNK,), jnp.float32),
          pltpu.VMEM((_CHUNK,), jnp.float32),
          pltpu.VMEM((_CHUNK,), jnp.float32),
          pltpu.SemaphoreType.DMA,
      ],
  )(_sc_sel_body)
  olab, ops_, opl = [o.reshape(B, _OUT_PAD) for o in sc_sel(
      lab, relationship.reshape(B * N * R),
      oidx.reshape(B * _OUT_PAD), oval.reshape(B * _OUT_PAD))]

  boxes_out = jnp.stack([ox1[:, :_MAX_DET], oy1[:, :_MAX_DET],
                         ox2[:, :_MAX_DET], oy2[:, :_MAX_DET]], axis=-1)
  return (boxes_out, osc[:, :_MAX_DET], olab[:, :_MAX_DET].astype(jnp.int32),
          ops_[:, :_MAX_DET], opl[:, :_MAX_DET].astype(jnp.int32))


# speedup vs baseline: 1.2392x; 1.0691x over previous
import functools

import jax
import jax.numpy as jnp
from jax import lax
from jax.experimental import pallas as pl
from jax.experimental.pallas import tpu as pltpu
from jax.experimental.pallas import tpu_sc as plsc

_NMS_THR = 0.5
_SCORE_THR = 0.05
_MAX_DET = 300
_NEG = -1e30
_OUT_PAD = 320
_CHUNK = 80


def _prep_body(cls_ref, box_ref, s_ref, lab_ref, x1_ref, y1_ref, x2_ref, y2_ref):
  cls = cls_ref[...]
  m = jnp.max(cls, axis=2)
  iota_c = lax.broadcasted_iota(jnp.int32, cls.shape, 2).astype(jnp.float32)
  lab = jnp.min(jnp.where(cls == m[:, :, None], iota_c, 1e9), axis=2)
  s_ref[...] = jnp.where(m > _SCORE_THR, m, _NEG)[:, None, None, :]
  lab_ref[...] = lab[:, None, None, :]
  box = box_ref[...]
  x1_ref[...] = box[:, None, None, :, 0]
  y1_ref[...] = box[:, None, None, :, 1]
  x2_ref[...] = box[:, None, None, :, 2]
  y2_ref[...] = box[:, None, None, :, 3]


def _nms_body(s_in_ref, x1_ref, y1_ref, x2_ref, y2_ref,
              ox1_ref, oy1_ref, ox2_ref, oy2_ref,
              osc_ref, oidx_ref, oval_ref,
              s_ref, area_ref):
  B, N = s_in_ref.shape
  s_ref[...] = s_in_ref[...]
  area_ref[...] = (x2_ref[...] - x1_ref[...]) * (y2_ref[...] - y1_ref[...])
  iota_n = lax.broadcasted_iota(jnp.int32, (B, N), 1).astype(jnp.float32)
  iota_o = lax.broadcasted_iota(jnp.int32, (B, _OUT_PAD), 1)
  for r in (ox1_ref, oy1_ref, ox2_ref, oy2_ref, osc_ref, oval_ref):
    r[...] = jnp.full((B, _OUT_PAD), -1.0, jnp.float32)
  oidx_ref[...] = jnp.zeros((B, _OUT_PAD), jnp.float32)

  def body(i, carry):
    s = s_ref[...]
    best = jnp.max(s, axis=1, keepdims=True)
    valid = best > (_NEG / 2)
    bidx = jnp.min(jnp.where(s == best, iota_n, 1e9), axis=1, keepdims=True)
    pos = iota_n == bidx

    def pick(a):
      return jnp.max(jnp.where(pos, a, -jnp.inf), axis=1, keepdims=True)

    x1, y1, x2, y2 = x1_ref[...], y1_ref[...], x2_ref[...], y2_ref[...]
    bx1, by1, bx2, by2 = pick(x1), pick(y1), pick(x2), pick(y2)
    barea = (bx2 - bx1) * (by2 - by1)

    xx1 = jnp.maximum(bx1, x1)
    yy1 = jnp.maximum(by1, y1)
    xx2 = jnp.minimum(bx2, x2)
    yy2 = jnp.minimum(by2, y2)
    inter = jnp.maximum(xx2 - xx1, 0.0) * jnp.maximum(yy2 - yy1, 0.0)
    area = area_ref[...]
    iou = inter / jnp.maximum(barea + area - inter, 1e-8)
    suppress = (iou > _NMS_THR) | pos
    s_ref[...] = jnp.where(suppress & valid, _NEG, s)

    col = iota_o == i
    colv = col & valid
    ox1_ref[...] = jnp.where(colv, bx1, ox1_ref[...])
    oy1_ref[...] = jnp.where(colv, by1, oy1_ref[...])
    ox2_ref[...] = jnp.where(colv, bx2, ox2_ref[...])
    oy2_ref[...] = jnp.where(colv, by2, oy2_ref[...])
    osc_ref[...] = jnp.where(colv, best, osc_ref[...])
    oidx_ref[...] = jnp.where(col, bidx, oidx_ref[...])
    oval_ref[...] = jnp.where(colv, 1.0, oval_ref[...])
    return carry

  lax.fori_loop(0, _MAX_DET, body, 0)


def _rowwise_arg_max(rows_v, ncols, r0, iota16):
  rows16 = iota16 + r0

  def col(j, st):
    m, marg = st
    g = plsc.load_gather(rows_v, [rows16, jnp.full((16,), 0, jnp.int32) + j])
    upd = g > m
    m = jnp.where(upd, g, m)
    marg = jnp.where(upd, jnp.full((16,), 0.0) + j.astype(jnp.float32), marg)
    return m, marg

  init = (jnp.full((16,), -jnp.inf, jnp.float32), jnp.zeros((16,), jnp.float32))
  return lax.fori_loop(0, ncols, col, init)


def _sc_sel_body(lab_hbm, rel_hbm, idx_hbm, val_hbm,
                 olab_hbm, ops_hbm, opl_hbm,
                 idx_v, val_v, idxg_v, idxe_v, plane_v, rbuf_v,
                 olab_v, ops_v, opl_v, sem):
  N = lab_hbm.shape[1]
  R = rel_hbm.shape[0] // (8 * N)
  wid = lax.axis_index("s") * 2 + lax.axis_index("c")
  b = wid // 4
  off = b * _OUT_PAD + (wid % 4) * _CHUNK
  iota16 = lax.iota(jnp.int32, 16)

  pltpu.sync_copy(idx_hbm.at[pl.ds(off, _CHUNK)], idx_v)
  pltpu.sync_copy(val_hbm.at[pl.ds(off, _CHUNK)], val_v)
  for j in range(0, _CHUNK, 16):
    iv = idx_v[pl.ds(j, 16)].astype(jnp.int32) + b * N
    idxg_v[pl.ds(j, 16)] = iv
  pltpu.sync_copy(lab_hbm.at[b], plane_v)

  def fill_idx(j, carry):
    for k in range(0, _CHUNK, 16):
      base = idxg_v[pl.ds(k, 16)] * R + j
      idxe_v[pl.ds(j * _CHUNK + k, 16)] = base
    return carry
  lax.fori_loop(0, R, fill_idx, 0)

  def fire(j, carry):
    pltpu.async_copy(rel_hbm.at[idxe_v.at[pl.ds(j * _CHUNK, _CHUNK)]],
                     rbuf_v.at[pl.ds(j * _CHUNK, _CHUNK)], sem)
    return carry
  lax.fori_loop(0, R, fire, 0)
  pltpu.make_async_copy(rel_hbm.at[pl.ds(0, R * _CHUNK)], rbuf_v, sem).wait()

  for r0 in range(0, _CHUNK, 16):
    vv = val_v[pl.ds(r0, 16)] > 0.0
    clab = plsc.load_gather(plane_v, [idx_v[pl.ds(r0, 16)].astype(jnp.int32)])

    def rcol(j, st):
      m, marg = st
      g = rbuf_v[pl.ds(j * _CHUNK + r0, 16)]
      upd = g > m
      m = jnp.where(upd, g, m)
      marg = jnp.where(upd, jnp.full((16,), 0.0) + j.astype(jnp.float32), marg)
      return m, marg
    rmax, rlab = lax.fori_loop(
        0, R, rcol,
        (jnp.full((16,), -jnp.inf, jnp.float32), jnp.zeros((16,), jnp.float32)))

    olab_v[pl.ds(r0, 16)] = jnp.where(vv, clab, -1.0)
    ops_v[pl.ds(r0, 16)] = jnp.where(vv, rmax, -1.0)
    opl_v[pl.ds(r0, 16)] = jnp.where(vv, rlab, -1.0)

  pltpu.sync_copy(olab_v, olab_hbm.at[pl.ds(off, _CHUNK)])
  pltpu.sync_copy(ops_v, ops_hbm.at[pl.ds(off, _CHUNK)])
  pltpu.sync_copy(opl_v, opl_hbm.at[pl.ds(off, _CHUNK)])


def kernel(boxes, classification, relationship):
  B, N, C = classification.shape
  R = relationship.shape[2]
  NB = 2000
  G = N // NB
  fbn = jax.ShapeDtypeStruct((B, G, 1, NB), jnp.float32)

  planes = pl.pallas_call(
      _prep_body,
      grid=(B, G),
      in_specs=[
          pl.BlockSpec((1, NB, C), lambda b, i: (b, i, 0)),
          pl.BlockSpec((1, NB, 4), lambda b, i: (b, i, 0)),
      ],
      out_specs=[pl.BlockSpec((1, 1, 1, NB), lambda b, i: (b, i, 0, 0))] * 6,
      out_shape=[fbn] * 6,
  )(classification, boxes)
  s, lab, x1, y1, x2, y2 = [p.reshape(B, N) for p in planes]

  fout = jax.ShapeDtypeStruct((B, _OUT_PAD), jnp.float32)
  nouts = pl.pallas_call(
      _nms_body,
      out_shape=[fout] * 7,
      scratch_shapes=[pltpu.VMEM((B, N), jnp.float32)] * 2,
  )(s, x1, y1, x2, y2)
  ox1, oy1, ox2, oy2, osc, oidx, oval = nouts

  mesh = plsc.VectorSubcoreMesh(core_axis_name="c", subcore_axis_name="s")
  sc_sel = functools.partial(
      pl.kernel,
      mesh=mesh,
      out_type=[jax.ShapeDtypeStruct((B * _OUT_PAD,), jnp.float32)] * 3,
      compiler_params=pltpu.CompilerParams(needs_layout_passes=False, use_tc_tiling_on_sc=False),
      scratch_types=[
          pltpu.VMEM((_CHUNK,), jnp.float32),
          pltpu.VMEM((_CHUNK,), jnp.float32),
          pltpu.VMEM((_CHUNK,), jnp.int32),
          pltpu.VMEM((R * _CHUNK,), jnp.int32),
          pltpu.VMEM((N,), jnp.float32),
          pltpu.VMEM((R * _CHUNK,), jnp.float32),
          pltpu.VMEM((_CHUNK,), jnp.float32),
          pltpu.VMEM((_CHUNK,), jnp.float32),
          pltpu.VMEM((_CHUNK,), jnp.float32),
          pltpu.SemaphoreType.DMA,
      ],
  )(_sc_sel_body)
  olab, ops_, opl = [o.reshape(B, _OUT_PAD) for o in sc_sel(
      lab, relationship.reshape(B * N * R),
      oidx.reshape(B * _OUT_PAD), oval.reshape(B * _OUT_PAD))]

  boxes_out = jnp.stack([ox1[:, :_MAX_DET], oy1[:, :_MAX_DET],
                         ox2[:, :_MAX_DET], oy2[:, :_MAX_DET]], axis=-1)
  return (boxes_out, osc[:, :_MAX_DET], olab[:, :_MAX_DET].astype(jnp.int32),
          ops_[:, :_MAX_DET], opl[:, :_MAX_DET].astype(jnp.int32))
